# baseline (speedup 1.0000x reference)
import functools

import jax
import jax.numpy as jnp
from jax import lax
from jax.experimental import pallas as pl
from jax.experimental.pallas import tpu as pltpu
from jax.experimental.pallas import tpu_sc as plsc

_N = 10000
_K = 32
_D = 128
_NPAD = 10240
_H = 2
_NC = _NPAD // _H
_BN = 80
_NW = 32
_NBUF = 4


def _sc_gather(emb, idx_nbr2, idx_u2):
    mesh = plsc.VectorSubcoreMesh(core_axis_name="c", subcore_axis_name="s")
    rows_per_w = _K * _NC // _NW // 128
    ngroups = rows_per_w // _NBUF
    u_rows = _NC // 128

    @functools.partial(
        pl.kernel,
        out_type=(
            jax.ShapeDtypeStruct((_K * _NC, _D), jnp.float32),
            jax.ShapeDtypeStruct((_NC, _D), jnp.float32),
        ),
        mesh=mesh,
        scratch_types=[
            pltpu.VMEM((rows_per_w, 1, 128), jnp.int32),
            pltpu.VMEM((1, 1, 128), jnp.int32),
            pltpu.VMEM((_NBUF, 128, _D), jnp.float32),
            pltpu.SemaphoreType.DMA,
            pltpu.SemaphoreType.DMA,
            pltpu.SemaphoreType.DMA,
            pltpu.SemaphoreType.DMA,
            pltpu.SemaphoreType.DMA,
            pltpu.SemaphoreType.DMA,
            pltpu.SemaphoreType.DMA,
            pltpu.SemaphoreType.DMA,
        ],
    )
    def gather_kernel(emb_hbm, idxn_hbm, idxu_hbm, nbr_out, u_out,
                      idx_all, idx_u, bufs,
                      g0, g1, g2, g3, w0, w1, w2, w3):
        gsem = (g0, g1, g2, g3)
        wsem = (w0, w1, w2, w3)
        wid = lax.axis_index("s") * 2 + lax.axis_index("c")
        base_row = wid * rows_per_w
        base_emb = wid * rows_per_w * 128

        pltpu.sync_copy(idxn_hbm.at[pl.ds(base_row, rows_per_w)], idx_all)

        def fire(c, b):
            pltpu.async_copy(emb_hbm.at[idx_all.at[c, 0]], bufs.at[b],
                             gsem[b])

        def wait_gather(b):
            pltpu.make_async_copy(
                emb_hbm.at[pl.ds(0, 128)], bufs.at[b], gsem[b]).wait()

        def write(c, b):
            pltpu.async_copy(
                bufs.at[b], nbr_out.at[pl.ds(base_emb + c * 128, 128)],
                wsem[b])

        def wait_write(b):
            pltpu.make_async_copy(
                bufs.at[b], nbr_out.at[pl.ds(0, 128)], wsem[b]).wait()

        for b in range(_NBUF):
            fire(b, b)

        def group(g, carry):
            for b in range(_NBUF):
                c = g * _NBUF + b
                wait_gather(b)
                write(c, b)

                @pl.when(c + _NBUF < rows_per_w)
                def _():
                    wait_write(b)
                    fire(c + _NBUF, b)
            return carry

        lax.fori_loop(0, ngroups, group, 0)
        for b in range(_NBUF):
            wait_write(b)

        for r in range(-(-u_rows // _NW)):
            row = wid + r * _NW

            @pl.when(row < u_rows)
            def _(row=row):
                pltpu.sync_copy(idxu_hbm.at[pl.ds(row, 1)], idx_u)
                cp = pltpu.async_copy(
                    emb_hbm.at[idx_u.at[0, 0]], bufs.at[0], g0)
                cp.wait()
                pltpu.sync_copy(bufs.at[0],
                                u_out.at[pl.ds(row * 128, 128)])

    return gather_kernel(emb, idx_nbr2, idx_u2)


def _tc_body(nbr_ref, u_ref, w1a_ref, w1b_ref, b1_ref, w2_ref, b2_ref,
             w3_ref, out_ref):
    ub = u_ref[...]
    t = jnp.dot(ub, w1b_ref[...],
                preferred_element_type=jnp.float32) + b1_ref[...]
    nbr = nbr_ref[...]
    nbr_flat = nbr.reshape(_K * _BN, _D)
    h = jnp.dot(nbr_flat, w1a_ref[...], preferred_element_type=jnp.float32)
    h = h + jnp.broadcast_to(t[None], (_K, _BN, _D)).reshape(_K * _BN, _D)
    h = jnp.maximum(h, 0.0)
    h2 = jnp.dot(h, w2_ref[...], preferred_element_type=jnp.float32)
    h2 = jnp.maximum(h2 + b2_ref[...], 0.0)
    h2_3 = h2.reshape(_K, _BN, _D)
    w3v = w3_ref[...]
    s = [jnp.sum(h2_3[k] * w3v, axis=1, keepdims=True) for k in range(_K)]
    m = functools.reduce(jnp.maximum, s)
    e = [jnp.exp(sk - m) for sk in s]
    den = functools.reduce(lambda a, b: a + b, e)
    acc = e[0] * nbr[0]
    for k in range(1, _K):
        acc = acc + e[k] * nbr[k]
    out_ref[...] = acc / den


def _tc_mlp(nbr_t, u, w1a, w1b, b1, w2, b2, w3r, interpret=False):
    nc = nbr_t.shape[1]
    return pl.pallas_call(
        _tc_body,
        grid=(nc // _BN,),
        in_specs=[
            pl.BlockSpec((_K, _BN, _D), lambda i: (0, i, 0)),
            pl.BlockSpec((_BN, _D), lambda i: (i, 0)),
            pl.BlockSpec((_D, _D), lambda i: (0, 0)),
            pl.BlockSpec((_D, _D), lambda i: (0, 0)),
            pl.BlockSpec((1, _D), lambda i: (0, 0)),
            pl.BlockSpec((_D, _D), lambda i: (0, 0)),
            pl.BlockSpec((1, _D), lambda i: (0, 0)),
            pl.BlockSpec((1, _D), lambda i: (0, 0)),
        ],
        out_specs=pl.BlockSpec((_BN, _D), lambda i: (i, 0)),
        out_shape=jax.ShapeDtypeStruct((nc, _D), jnp.float32),
        interpret=interpret,
    )(nbr_t, u, w1a, w1b, b1, w2, b2, w3r)


def kernel(nodes, neighbor_nodes, emb, W1, b1, W2, b2, W3, b3):
    pad = jnp.arange(_NPAD - _N, dtype=neighbor_nodes.dtype)
    nbr_pad = jnp.concatenate(
        [neighbor_nodes, jnp.broadcast_to(pad[:, None], (_NPAD - _N, _K))],
        axis=0)
    nbr_t_idx = jnp.transpose(nbr_pad.reshape(_H, _NC, _K), (0, 2, 1))
    idx_nbr = nbr_t_idx.reshape(_H, -1, 1, 128)
    idx_u = jnp.concatenate([nodes, pad]).reshape(_H, -1, 1, 128)
    w1a, w1b = W1[:_D], W1[_D:]
    b1r, b2r, w3r = b1.reshape(1, _D), b2.reshape(1, _D), W3.reshape(1, _D)
    outs = []
    for h in range(_H):
        nbr_flat, u = _sc_gather(emb, idx_nbr[h], idx_u[h])
        outs.append(_tc_mlp(nbr_flat.reshape(_K, _NC, _D), u,
                            w1a, w1b, b1r, W2, b2r, w3r))
    return jnp.concatenate(outs, axis=0)[:_N]


# speedup vs baseline: 1.0382x; 1.0382x over previous
import functools

import jax
import jax.numpy as jnp
from jax import lax
from jax.experimental import pallas as pl
from jax.experimental.pallas import tpu as pltpu
from jax.experimental.pallas import tpu_sc as plsc

_N = 10000
_K = 32
_D = 128
_NPAD = 10240
_H = 4
_NC = _NPAD // _H
_BN = 80
_NW = 32
_NBUF = 4


def _sc_gather(emb, idx_nbr2, idx_u2):
    mesh = plsc.VectorSubcoreMesh(core_axis_name="c", subcore_axis_name="s")
    rows_per_w = _K * _NC // _NW // 128
    ngroups = rows_per_w // _NBUF
    u_rows = _NC // 128

    @functools.partial(
        pl.kernel,
        out_type=(
            jax.ShapeDtypeStruct((_K * _NC, _D), jnp.float32),
            jax.ShapeDtypeStruct((_NC, _D), jnp.float32),
        ),
        mesh=mesh,
        scratch_types=[
            pltpu.VMEM((rows_per_w, 1, 128), jnp.int32),
            pltpu.VMEM((1, 1, 128), jnp.int32),
            pltpu.VMEM((_NBUF, 128, _D), jnp.float32),
            pltpu.SemaphoreType.DMA,
            pltpu.SemaphoreType.DMA,
            pltpu.SemaphoreType.DMA,
            pltpu.SemaphoreType.DMA,
            pltpu.SemaphoreType.DMA,
            pltpu.SemaphoreType.DMA,
            pltpu.SemaphoreType.DMA,
            pltpu.SemaphoreType.DMA,
        ],
    )
    def gather_kernel(emb_hbm, idxn_hbm, idxu_hbm, nbr_out, u_out,
                      idx_all, idx_u, bufs,
                      g0, g1, g2, g3, w0, w1, w2, w3):
        gsem = (g0, g1, g2, g3)
        wsem = (w0, w1, w2, w3)
        wid = lax.axis_index("s") * 2 + lax.axis_index("c")
        base_row = wid * rows_per_w
        base_emb = wid * rows_per_w * 128

        pltpu.sync_copy(idxn_hbm.at[pl.ds(base_row, rows_per_w)], idx_all)

        def fire(c, b):
            pltpu.async_copy(emb_hbm.at[idx_all.at[c, 0]], bufs.at[b],
                             gsem[b])

        def wait_gather(b):
            pltpu.make_async_copy(
                emb_hbm.at[pl.ds(0, 128)], bufs.at[b], gsem[b]).wait()

        def write(c, b):
            pltpu.async_copy(
                bufs.at[b], nbr_out.at[pl.ds(base_emb + c * 128, 128)],
                wsem[b])

        def wait_write(b):
            pltpu.make_async_copy(
                bufs.at[b], nbr_out.at[pl.ds(0, 128)], wsem[b]).wait()

        for b in range(_NBUF):
            fire(b, b)

        def group(g, carry):
            for b in range(_NBUF):
                c = g * _NBUF + b
                wait_gather(b)
                write(c, b)

                @pl.when(c + _NBUF < rows_per_w)
                def _():
                    wait_write(b)
                    fire(c + _NBUF, b)
            return carry

        lax.fori_loop(0, ngroups, group, 0)
        for b in range(_NBUF):
            wait_write(b)

        for r in range(-(-u_rows // _NW)):
            row = wid + r * _NW

            @pl.when(row < u_rows)
            def _(row=row):
                pltpu.sync_copy(idxu_hbm.at[pl.ds(row, 1)], idx_u)
                cp = pltpu.async_copy(
                    emb_hbm.at[idx_u.at[0, 0]], bufs.at[0], g0)
                cp.wait()
                pltpu.sync_copy(bufs.at[0],
                                u_out.at[pl.ds(row * 128, 128)])

    return gather_kernel(emb, idx_nbr2, idx_u2)


def _tc_body(nbr_ref, u_ref, w1a_ref, w1b_ref, b1_ref, w2_ref, b2_ref,
             w3_ref, out_ref):
    ub = u_ref[...]
    t = jnp.dot(ub, w1b_ref[...],
                preferred_element_type=jnp.float32) + b1_ref[...]
    nbr = nbr_ref[...]
    nbr_flat = nbr.reshape(_K * _BN, _D)
    h = jnp.dot(nbr_flat, w1a_ref[...], preferred_element_type=jnp.float32)
    h = h + jnp.broadcast_to(t[None], (_K, _BN, _D)).reshape(_K * _BN, _D)
    h = jnp.maximum(h, 0.0)
    h2 = jnp.dot(h, w2_ref[...], preferred_element_type=jnp.float32)
    h2 = jnp.maximum(h2 + b2_ref[...], 0.0)
    h2_3 = h2.reshape(_K, _BN, _D)
    w3v = w3_ref[...]
    s = [jnp.sum(h2_3[k] * w3v, axis=1, keepdims=True) for k in range(_K)]
    m = functools.reduce(jnp.maximum, s)
    e = [jnp.exp(sk - m) for sk in s]
    den = functools.reduce(lambda a, b: a + b, e)
    acc = e[0] * nbr[0]
    for k in range(1, _K):
        acc = acc + e[k] * nbr[k]
    out_ref[...] = acc / den


def _tc_mlp(nbr_t, u, w1a, w1b, b1, w2, b2, w3r, interpret=False):
    nc = nbr_t.shape[1]
    return pl.pallas_call(
        _tc_body,
        grid=(nc // _BN,),
        in_specs=[
            pl.BlockSpec((_K, _BN, _D), lambda i: (0, i, 0)),
            pl.BlockSpec((_BN, _D), lambda i: (i, 0)),
            pl.BlockSpec((_D, _D), lambda i: (0, 0)),
            pl.BlockSpec((_D, _D), lambda i: (0, 0)),
            pl.BlockSpec((1, _D), lambda i: (0, 0)),
            pl.BlockSpec((_D, _D), lambda i: (0, 0)),
            pl.BlockSpec((1, _D), lambda i: (0, 0)),
            pl.BlockSpec((1, _D), lambda i: (0, 0)),
        ],
        out_specs=pl.BlockSpec((_BN, _D), lambda i: (i, 0)),
        out_shape=jax.ShapeDtypeStruct((nc, _D), jnp.float32),
        interpret=interpret,
    )(nbr_t, u, w1a, w1b, b1, w2, b2, w3r)


def kernel(nodes, neighbor_nodes, emb, W1, b1, W2, b2, W3, b3):
    pad = jnp.arange(_NPAD - _N, dtype=neighbor_nodes.dtype)
    nbr_pad = jnp.concatenate(
        [neighbor_nodes, jnp.broadcast_to(pad[:, None], (_NPAD - _N, _K))],
        axis=0)
    nbr_t_idx = jnp.transpose(nbr_pad.reshape(_H, _NC, _K), (0, 2, 1))
    idx_nbr = nbr_t_idx.reshape(_H, -1, 1, 128)
    idx_u = jnp.concatenate([nodes, pad]).reshape(_H, -1, 1, 128)
    w1a, w1b = W1[:_D], W1[_D:]
    b1r, b2r, w3r = b1.reshape(1, _D), b2.reshape(1, _D), W3.reshape(1, _D)
    gathered = [_sc_gather(emb, idx_nbr[h], idx_u[h]) for h in range(_H)]
    outs = [_tc_mlp(nbr_flat.reshape(_K, _NC, _D), u,
                    w1a, w1b, b1r, W2, b2r, w3r)
            for nbr_flat, u in gathered]
    return jnp.concatenate(outs, axis=0)[:_N]


# speedup vs baseline: 1.0390x; 1.0009x over previous
import functools

import jax
import jax.numpy as jnp
from jax import lax
from jax.experimental import pallas as pl
from jax.experimental.pallas import tpu as pltpu
from jax.experimental.pallas import tpu_sc as plsc

_N = 10000
_K = 32
_D = 128
_NPAD = 10240
_H = 4
_NC = _NPAD // _H
_BN = 80
_NW = 32
_NBUF = 4


def _sc_gather(emb, idx_nbr2, idx_u2):
    mesh = plsc.VectorSubcoreMesh(core_axis_name="c", subcore_axis_name="s")
    rows_per_w = _K * _NC // _NW // 128
    ngroups = rows_per_w // _NBUF
    u_rows = _NC // 128

    @functools.partial(
        pl.kernel,
        out_type=(
            jax.ShapeDtypeStruct((_K * _NC, _D), jnp.float32),
            jax.ShapeDtypeStruct((_NC, _D), jnp.float32),
        ),
        mesh=mesh,
        scratch_types=[
            pltpu.VMEM((rows_per_w, 1, 128), jnp.int32),
            pltpu.VMEM((1, 1, 128), jnp.int32),
            pltpu.VMEM((_NBUF, 128, _D), jnp.float32),
            pltpu.SemaphoreType.DMA,
            pltpu.SemaphoreType.DMA,
            pltpu.SemaphoreType.DMA,
            pltpu.SemaphoreType.DMA,
            pltpu.SemaphoreType.DMA,
            pltpu.SemaphoreType.DMA,
            pltpu.SemaphoreType.DMA,
            pltpu.SemaphoreType.DMA,
        ],
    )
    def gather_kernel(emb_hbm, idxn_hbm, idxu_hbm, nbr_out, u_out,
                      idx_all, idx_u, bufs,
                      g0, g1, g2, g3, w0, w1, w2, w3):
        gsem = (g0, g1, g2, g3)
        wsem = (w0, w1, w2, w3)
        wid = lax.axis_index("s") * 2 + lax.axis_index("c")
        base_row = wid * rows_per_w
        base_emb = wid * rows_per_w * 128

        pltpu.sync_copy(idxn_hbm.at[pl.ds(base_row, rows_per_w)], idx_all)

        def fire(c, b):
            pltpu.async_copy(emb_hbm.at[idx_all.at[c, 0]], bufs.at[b],
                             gsem[b])

        def wait_gather(b):
            pltpu.make_async_copy(
                emb_hbm.at[pl.ds(0, 128)], bufs.at[b], gsem[b]).wait()

        def write(c, b):
            pltpu.async_copy(
                bufs.at[b], nbr_out.at[pl.ds(base_emb + c * 128, 128)],
                wsem[b])

        def wait_write(b):
            pltpu.make_async_copy(
                bufs.at[b], nbr_out.at[pl.ds(0, 128)], wsem[b]).wait()

        for b in range(_NBUF):
            fire(b, b)

        def group(g, carry):
            for b in range(_NBUF):
                c = g * _NBUF + b
                wait_gather(b)
                write(c, b)

                @pl.when(c + _NBUF < rows_per_w)
                def _():
                    wait_write(b)
                    fire(c + _NBUF, b)
            return carry

        lax.fori_loop(0, ngroups, group, 0)
        for b in range(_NBUF):
            wait_write(b)

        for r in range(-(-u_rows // _NW)):
            row = wid + r * _NW

            @pl.when(row < u_rows)
            def _(row=row):
                pltpu.sync_copy(idxu_hbm.at[pl.ds(row, 1)], idx_u)
                cp = pltpu.async_copy(
                    emb_hbm.at[idx_u.at[0, 0]], bufs.at[0], g0)
                cp.wait()
                pltpu.sync_copy(bufs.at[0],
                                u_out.at[pl.ds(row * 128, 128)])

    return gather_kernel(emb, idx_nbr2, idx_u2)


def _tc_body(nbr_ref, u_ref, w1a_ref, w1b_ref, b1_ref, w2_ref, b2_ref,
             w3_ref, out_ref):
    ub = u_ref[...]
    t = jnp.dot(ub, w1b_ref[...],
                preferred_element_type=jnp.float32) + b1_ref[...]
    nbr = nbr_ref[...]
    nbr_flat = nbr.reshape(_K * _BN, _D)
    h = jnp.dot(nbr_flat.astype(jnp.bfloat16), w1a_ref[...],
                preferred_element_type=jnp.float32)
    h = h + jnp.broadcast_to(t[None], (_K, _BN, _D)).reshape(_K * _BN, _D)
    h = jnp.maximum(h, 0.0).astype(jnp.bfloat16)
    h2 = jnp.dot(h, w2_ref[...], preferred_element_type=jnp.float32)
    h2 = jnp.maximum(h2 + b2_ref[...], 0.0)
    h2_3 = h2.reshape(_K, _BN, _D)
    w3v = w3_ref[...]
    s = [jnp.sum(h2_3[k] * w3v, axis=1, keepdims=True) for k in range(_K)]
    m = functools.reduce(jnp.maximum, s)
    e = [jnp.exp(sk - m) for sk in s]
    den = functools.reduce(lambda a, b: a + b, e)
    acc = e[0] * nbr[0]
    for k in range(1, _K):
        acc = acc + e[k] * nbr[k]
    out_ref[...] = acc / den


def _tc_mlp(nbr_t, u, w1a, w1b, b1, w2, b2, w3r, interpret=False):
    nc = nbr_t.shape[1]
    return pl.pallas_call(
        _tc_body,
        grid=(nc // _BN,),
        in_specs=[
            pl.BlockSpec((_K, _BN, _D), lambda i: (0, i, 0)),
            pl.BlockSpec((_BN, _D), lambda i: (i, 0)),
            pl.BlockSpec((_D, _D), lambda i: (0, 0)),
            pl.BlockSpec((_D, _D), lambda i: (0, 0)),
            pl.BlockSpec((1, _D), lambda i: (0, 0)),
            pl.BlockSpec((_D, _D), lambda i: (0, 0)),
            pl.BlockSpec((1, _D), lambda i: (0, 0)),
            pl.BlockSpec((1, _D), lambda i: (0, 0)),
        ],
        out_specs=pl.BlockSpec((_BN, _D), lambda i: (i, 0)),
        out_shape=jax.ShapeDtypeStruct((nc, _D), jnp.float32),
        interpret=interpret,
    )(nbr_t, u, w1a, w1b, b1, w2, b2, w3r)


def kernel(nodes, neighbor_nodes, emb, W1, b1, W2, b2, W3, b3):
    pad = jnp.arange(_NPAD - _N, dtype=neighbor_nodes.dtype)
    nbr_pad = jnp.concatenate(
        [neighbor_nodes, jnp.broadcast_to(pad[:, None], (_NPAD - _N, _K))],
        axis=0)
    nbr_t_idx = jnp.transpose(nbr_pad.reshape(_H, _NC, _K), (0, 2, 1))
    idx_nbr = nbr_t_idx.reshape(_H, -1, 1, 128)
    idx_u = jnp.concatenate([nodes, pad]).reshape(_H, -1, 1, 128)
    w1a, w1b = W1[:_D].astype(jnp.bfloat16), W1[_D:]
    w2c = W2.astype(jnp.bfloat16)
    b1r, b2r, w3r = b1.reshape(1, _D), b2.reshape(1, _D), W3.reshape(1, _D)
    gathered = [_sc_gather(emb, idx_nbr[h], idx_u[h]) for h in range(_H)]
    outs = [_tc_mlp(nbr_flat.reshape(_K, _NC, _D), u,
                    w1a, w1b, b1r, w2c, b2r, w3r)
            for nbr_flat, u in gathered]
    return jnp.concatenate(outs, axis=0)[:_N]
